# baseline (speedup 1.0000x reference)
import functools
import jax
import jax.numpy as jnp
from jax import lax
from jax.experimental import pallas as pl
from jax.experimental.pallas import tpu as pltpu
from jax.experimental.pallas import tpu_sc as plsc

_B = 16384
_E = 64
_NC = 2
_NS = 16
_NW = _NC * _NS
_BPW = _B // _NW
_PH = 256
_CH = 128
_TBLK = 512


def _transpose_pad(xt, n_rows):
    def body(x_ref, y_ref):
        x = x_ref[...]
        y_ref[:, 0:_E] = jnp.transpose(x)
        y_ref[:, _E:128] = jnp.zeros((_TBLK, _E), jnp.float32)

    nb = pl.cdiv(n_rows, _TBLK)
    return pl.pallas_call(
        body,
        grid=(nb,),
        in_specs=[pl.BlockSpec((_E, _TBLK), lambda b: (0, b))],
        out_specs=pl.BlockSpec((_TBLK, 128), lambda b: (b, 0)),
        out_shape=jax.ShapeDtypeStruct((n_rows, 128), jnp.float32),
    )(xt)


def _make_sc_kernel():
    mesh = plsc.VectorSubcoreMesh(core_axis_name="c", subcore_axis_name="s")

    @functools.partial(
        pl.kernel,
        mesh=mesh,
        out_type=jax.ShapeDtypeStruct((_B,), jnp.float32),
        compiler_params=pltpu.CompilerParams(needs_layout_passes=False),
        scratch_types=[
            pltpu.VMEM((4, _CH), jnp.int32),
            pltpu.VMEM((4, _CH), jnp.int32),
            pltpu.VMEM((4, _CH), jnp.int32),
            pltpu.VMEM((_PH, 128), jnp.float32),
            pltpu.VMEM((_PH, 128), jnp.float32),
            pltpu.VMEM((_PH, 128), jnp.float32),
            pltpu.VMEM((_BPW,), jnp.float32),
            pltpu.SemaphoreType.DMA,
        ],
    )
    def lp_kernel(s_hbm, p_hbm, o_hbm, ent_hbm, rel_hbm, out_hbm,
                  s_idx, p_idx, o_idx, s_rows, p_rows, o_rows, out_v, sem):
        wid = lax.axis_index("s") * _NC + lax.axis_index("c")
        base = wid * _BPW

        for j in range(4):
            pltpu.sync_copy(s_hbm.at[pl.ds(base + j * _CH, _CH)], s_idx.at[j])
            pltpu.sync_copy(p_hbm.at[pl.ds(base + j * _CH, _CH)], p_idx.at[j])
            pltpu.sync_copy(o_hbm.at[pl.ds(base + j * _CH, _CH)], o_idx.at[j])

        lane = lax.iota(jnp.int32, 16)

        for ph in range(2):
            copies = []
            for j in range(2):
                rows = pl.ds(j * _CH, _CH)
                jj = ph * 2 + j
                copies.append(pltpu.async_copy(ent_hbm.at[s_idx.at[jj]], s_rows.at[rows], sem))
                copies.append(pltpu.async_copy(rel_hbm.at[p_idx.at[jj]], p_rows.at[rows], sem))
                copies.append(pltpu.async_copy(ent_hbm.at[o_idx.at[jj]], o_rows.at[rows], sem))
            for c in copies:
                c.wait()

            def chunk_body(ci, carry):
                row_ids = ci * 16 + lane
                acc = jnp.zeros((16,), jnp.float32)
                for e in range(_E):
                    col = jnp.full((16,), e, dtype=jnp.int32)
                    a = plsc.load_gather(s_rows, [row_ids, col])
                    b = plsc.load_gather(p_rows, [row_ids, col])
                    c = plsc.load_gather(o_rows, [row_ids, col])
                    acc = acc + a * b * c
                out_v[pl.ds(ph * _PH + ci * 16, 16)] = acc
                return carry

            lax.fori_loop(0, _PH // 16, chunk_body, 0)

        pltpu.sync_copy(out_v, out_hbm.at[pl.ds(base, _BPW)])

    return lp_kernel


_lp_kernel = None


def kernel(s, p, o, entities, relations):
    global _lp_kernel
    if _lp_kernel is None:
        _lp_kernel = _make_sc_kernel()
    ent_pad = _transpose_pad(jnp.swapaxes(entities, 0, 1), entities.shape[0])
    rel_pad = _transpose_pad(jnp.swapaxes(relations, 0, 1), relations.shape[0])
    return _lp_kernel(s, p, o, ent_pad, rel_pad)


# speedup vs baseline: 2.2479x; 2.2479x over previous
import functools
import jax
import jax.numpy as jnp
from jax import lax
from jax.experimental import pallas as pl
from jax.experimental.pallas import tpu as pltpu
from jax.experimental.pallas import tpu_sc as plsc

_B = 16384
_E = 64
_NC = 2
_NS = 16
_NW = _NC * _NS
_BPW = _B // _NW
_PH = 256
_CH = 128
_TBLK = 2048


def _transpose_pad(xt, n_rows):
    def body(x_ref, y_ref):
        x = x_ref[...]
        eye = jnp.eye(_E, dtype=jnp.float32)
        y_ref[:, 0:_E] = jax.lax.dot_general(
            x, eye, (((0,), (0,)), ((), ())),
            preferred_element_type=jnp.float32)
        y_ref[:, _E:128] = jnp.zeros((_TBLK, _E), jnp.float32)

    nb = pl.cdiv(n_rows, _TBLK)
    return pl.pallas_call(
        body,
        grid=(nb,),
        in_specs=[pl.BlockSpec((_E, _TBLK), lambda b: (0, b))],
        out_specs=pl.BlockSpec((_TBLK, 128), lambda b: (b, 0)),
        out_shape=jax.ShapeDtypeStruct((n_rows, 128), jnp.float32),
    )(xt)


def _make_sc_kernel():
    mesh = plsc.VectorSubcoreMesh(core_axis_name="c", subcore_axis_name="s")

    @functools.partial(
        pl.kernel,
        mesh=mesh,
        out_type=jax.ShapeDtypeStruct((_B,), jnp.float32),
        compiler_params=pltpu.CompilerParams(needs_layout_passes=False),
        scratch_types=[
            pltpu.VMEM((4, _CH), jnp.int32),
            pltpu.VMEM((4, _CH), jnp.int32),
            pltpu.VMEM((4, _CH), jnp.int32),
            pltpu.VMEM((_PH, 128), jnp.float32),
            pltpu.VMEM((_PH, 128), jnp.float32),
            pltpu.VMEM((_PH, 128), jnp.float32),
            pltpu.VMEM((_BPW,), jnp.float32),
            pltpu.SemaphoreType.DMA,
        ],
    )
    def lp_kernel(s_hbm, p_hbm, o_hbm, ent_hbm, rel_hbm, out_hbm,
                  s_idx, p_idx, o_idx, s_rows, p_rows, o_rows, out_v, sem):
        wid = lax.axis_index("s") * _NC + lax.axis_index("c")
        base = wid * _BPW

        for j in range(4):
            pltpu.sync_copy(s_hbm.at[pl.ds(base + j * _CH, _CH)], s_idx.at[j])
            pltpu.sync_copy(p_hbm.at[pl.ds(base + j * _CH, _CH)], p_idx.at[j])
            pltpu.sync_copy(o_hbm.at[pl.ds(base + j * _CH, _CH)], o_idx.at[j])

        lane = lax.iota(jnp.int32, 16)

        for ph in range(2):
            copies = []
            for j in range(2):
                rows = pl.ds(j * _CH, _CH)
                jj = ph * 2 + j
                copies.append(pltpu.async_copy(ent_hbm.at[s_idx.at[jj]], s_rows.at[rows], sem))
                copies.append(pltpu.async_copy(rel_hbm.at[p_idx.at[jj]], p_rows.at[rows], sem))
                copies.append(pltpu.async_copy(ent_hbm.at[o_idx.at[jj]], o_rows.at[rows], sem))
            for c in copies:
                c.wait()

            def chunk_body(ci, carry):
                row_ids = ci * 16 + lane
                acc = jnp.zeros((16,), jnp.float32)
                for e in range(_E):
                    col = jnp.full((16,), e, dtype=jnp.int32)
                    a = plsc.load_gather(s_rows, [row_ids, col])
                    b = plsc.load_gather(p_rows, [row_ids, col])
                    c = plsc.load_gather(o_rows, [row_ids, col])
                    acc = acc + a * b * c
                out_v[pl.ds(ph * _PH + ci * 16, 16)] = acc
                return carry

            lax.fori_loop(0, _PH // 16, chunk_body, 0)

        pltpu.sync_copy(out_v, out_hbm.at[pl.ds(base, _BPW)])

    return lp_kernel


_lp_kernel = None


def kernel(s, p, o, entities, relations):
    global _lp_kernel
    if _lp_kernel is None:
        _lp_kernel = _make_sc_kernel()
    ent_pad = _transpose_pad(jnp.swapaxes(entities, 0, 1), entities.shape[0])
    rel_pad = _transpose_pad(jnp.swapaxes(relations, 0, 1), relations.shape[0])
    return _lp_kernel(s, p, o, ent_pad, rel_pad)


# speedup vs baseline: 3.7222x; 1.6559x over previous
import functools
import jax
import jax.numpy as jnp
from jax import lax
from jax.experimental import pallas as pl
from jax.experimental.pallas import tpu as pltpu
from jax.experimental.pallas import tpu_sc as plsc

_B = 16384
_E = 64
_NC = 2
_NS = 16
_NW = _NC * _NS
_BPW = _B // _NW
_PH = 128
_NPH = _BPW // _PH
_TBLK = 8192


def _transpose_pad(xt, n_rows):
    def body(x_ref, y_ref):
        xi = jax.lax.bitcast_convert_type(x_ref[...], jnp.int32)
        yi = jnp.transpose(xi)
        y_ref[:, 0:_E] = jax.lax.bitcast_convert_type(yi, jnp.float32)

    nb = pl.cdiv(n_rows, _TBLK)
    return pl.pallas_call(
        body,
        grid=(nb,),
        compiler_params=pltpu.CompilerParams(
            dimension_semantics=("arbitrary",),
        ),
        in_specs=[pl.BlockSpec((_E, _TBLK), lambda b: (0, b))],
        out_specs=pl.BlockSpec((_TBLK, 128), lambda b: (b, 0)),
        out_shape=jax.ShapeDtypeStruct((n_rows, 128), jnp.float32),
    )(xt)


def _make_sc_kernel():
    mesh = plsc.VectorSubcoreMesh(core_axis_name="c", subcore_axis_name="s")

    @functools.partial(
        pl.kernel,
        mesh=mesh,
        out_type=jax.ShapeDtypeStruct((_B,), jnp.float32),
        compiler_params=pltpu.CompilerParams(needs_layout_passes=False),
        scratch_types=[
            pltpu.VMEM((_BPW,), jnp.int32),
            pltpu.VMEM((_BPW,), jnp.int32),
            pltpu.VMEM((_BPW,), jnp.int32),
            pltpu.VMEM((_PH, 128), jnp.float32),
            pltpu.VMEM((_PH, 128), jnp.float32),
            pltpu.VMEM((_PH, 128), jnp.float32),
            pltpu.VMEM((_PH, 128), jnp.float32),
            pltpu.VMEM((_PH, 128), jnp.float32),
            pltpu.VMEM((_PH, 128), jnp.float32),
            pltpu.VMEM((_BPW,), jnp.float32),
            pltpu.SemaphoreType.DMA,
            pltpu.SemaphoreType.DMA,
        ],
    )
    def lp_kernel(s_hbm, p_hbm, o_hbm, ent_hbm, rel_hbm, out_hbm,
                  s_idx, p_idx, o_idx, s0, s1, p0, p1, o0, o1,
                  out_v, semA, semB):
        wid = lax.axis_index("s") * _NC + lax.axis_index("c")
        base = wid * _BPW

        ic = [
            pltpu.async_copy(s_hbm.at[pl.ds(base, _BPW)], s_idx, semA),
            pltpu.async_copy(p_hbm.at[pl.ds(base, _BPW)], p_idx, semA),
            pltpu.async_copy(o_hbm.at[pl.ds(base, _BPW)], o_idx, semA),
        ]
        for c in ic:
            c.wait()

        sbuf = [s0, s1]
        pbuf = [p0, p1]
        obuf = [o0, o1]
        sems = [semA, semB]
        lane = lax.iota(jnp.int32, 16)

        def fire(ph):
            k = ph % 2
            sl = pl.ds(ph * _PH, _PH)
            sem = sems[k]
            return [
                pltpu.async_copy(ent_hbm.at[s_idx.at[sl]], sbuf[k], sem),
                pltpu.async_copy(rel_hbm.at[p_idx.at[sl]], pbuf[k], sem),
                pltpu.async_copy(ent_hbm.at[o_idx.at[sl]], obuf[k], sem),
            ]

        pend = fire(0)
        for ph in range(_NPH):
            cur = pend
            if ph + 1 < _NPH:
                pend = fire(ph + 1)
            for c in cur:
                c.wait()
            k = ph % 2
            sb, pb, ob = sbuf[k], pbuf[k], obuf[k]

            def chunk_body(ci, carry):
                row_ids = ci * 16 + lane
                acc = jnp.zeros((16,), jnp.float32)
                for e in range(_E):
                    col = jnp.full((16,), e, dtype=jnp.int32)
                    a = plsc.load_gather(sb, [row_ids, col])
                    b = plsc.load_gather(pb, [row_ids, col])
                    c = plsc.load_gather(ob, [row_ids, col])
                    acc = acc + a * b * c
                out_v[pl.ds(ph * _PH + ci * 16, 16)] = acc
                return carry

            lax.fori_loop(0, _PH // 16, chunk_body, 0)

        pltpu.sync_copy(out_v, out_hbm.at[pl.ds(base, _BPW)])

    return lp_kernel


_lp_kernel = None


def kernel(s, p, o, entities, relations):
    global _lp_kernel
    if _lp_kernel is None:
        _lp_kernel = _make_sc_kernel()
    ent_pad = _transpose_pad(jnp.swapaxes(entities, 0, 1), entities.shape[0])
    rel_pad = _transpose_pad(jnp.swapaxes(relations, 0, 1), relations.shape[0])
    return _lp_kernel(s, p, o, ent_pad, rel_pad)
